# baseline (speedup 1.0000x reference)
import jax
import jax.numpy as jnp
from jax.experimental import pallas as pl
from jax.experimental.pallas import tpu as pltpu

_BM = 128
_NSPLIT = 4


def _matmul_body(xa_ref, xb_ref, xc_ref, xd_ref, w_ref, b_ref, o_ref):
    wt = w_ref[...]
    bb = b_ref[...]
    for s, x_ref in enumerate((xa_ref, xb_ref, xc_ref, xd_ref)):
        o_ref[pl.ds(s * _BM, _BM), :] = jax.lax.dot_general(
            x_ref[...], wt,
            dimension_numbers=(((1,), (1,)), ((), ())),
            preferred_element_type=jnp.float32,
        ) + bb


@jax.jit
def kernel(input, weight, bias):
    m, k = input.shape
    n = weight.shape[0]
    grid = (m // (_NSPLIT * _BM),)

    def xspec(s):
        return pl.BlockSpec((_BM, k), lambda i: (_NSPLIT * i + s, 0))

    return pl.pallas_call(
        _matmul_body,
        grid=grid,
        in_specs=[
            xspec(0), xspec(1), xspec(2), xspec(3),
            pl.BlockSpec((n, k), lambda i: (0, 0)),
            pl.BlockSpec((1, n), lambda i: (0, 0)),
        ],
        out_specs=pl.BlockSpec((_NSPLIT * _BM, n), lambda i: (i, 0)),
        out_shape=jax.ShapeDtypeStruct((m, n), jnp.float32),
        compiler_params=pltpu.CompilerParams(
            dimension_semantics=("parallel",),
        ),
    )(input, input, input, input, weight, bias.reshape(1, n))
